# baseline (speedup 1.0000x reference)
import functools

import jax
import jax.numpy as jnp
from jax import lax
from jax.experimental import pallas as pl
from jax.experimental.pallas import tpu as pltpu
from jax.experimental.pallas import tpu_sc as plsc

C, H, W = 768, 224, 224
HW = H * W
_HALF = HW // 2

_NC = 2
_NS = 16
_NW = _NC * _NS
_CPW = C // _NW
_PAIRS = _CPW // 2

_CHUNK = 896
_NCHK = HW // _CHUNK
_NOB = 4
_KGRP = _NCHK // _NOB
_BLKS = _CHUNK // 32


def _row_copies(img_hbm, ch, row, sema, semb):
    ca = pltpu.make_async_copy(
        img_hbm.at[ch, pl.ds(0, _HALF)], row.at[pl.ds(0, _HALF)], sema)
    cb = pltpu.make_async_copy(
        img_hbm.at[ch, pl.ds(_HALF, _HALF)], row.at[pl.ds(_HALF, _HALF)], semb)
    return ca, cb


def _shuffle_body(img_hbm, idxp_hbm, out_hbm, idx_v, row0, row1,
                  ob0, ob1, ob2, ob3,
                  sg0a, sg0b, sg1a, sg1b, so0, so1, so2, so3):
    rows = (row0, row1)
    semg = ((sg0a, sg0b), (sg1a, sg1b))
    outb = (ob0, ob1, ob2, ob3)
    semo = (so0, so1, so2, so3)
    wid = lax.axis_index("s") * _NC + lax.axis_index("c")
    base_ch = wid * _CPW

    pltpu.sync_copy(idxp_hbm, idx_v)

    for p in (0, 1):
        for c in _row_copies(img_hbm, base_ch + p, rows[p], *semg[p]):
            c.start()

    def pair_body(g, carry):
        for p in (0, 1):
            ch = base_ch + 2 * g + p
            row = rows[p]
            for c in _row_copies(img_hbm, ch, row, *semg[p]):
                c.wait()

            def chunk_grp(k, carry2, p=p, ch=ch, row=row, g=g):
                for b in range(_NOB):
                    ck = _NOB * k + b

                    def do_wait(b=b, ch=ch, ck=ck):
                        pltpu.make_async_copy(
                            outb[b],
                            out_hbm.at[ch, pl.ds(ck * _CHUNK, _CHUNK)],
                            semo[b],
                        ).wait()

                    if p == 0:
                        pl.when(jnp.logical_or(g > 0, k > 0))(do_wait)
                    else:
                        do_wait()

                    @plsc.parallel_loop(0, _BLKS, unroll=7)
                    def _(t, b=b, ck=ck, row=row):
                        jbase = ck * (_CHUNK // 2) + 16 * t
                        v = idx_v[pl.ds(jbase, 16)]
                        lo = v & 0xFFFF
                        hi = (v >> 16) & 0xFFFF
                        outb[b][pl.ds(32 * t, 16)] = plsc.load_gather(
                            row, [lo]
                        )
                        outb[b][pl.ds(32 * t + 16, 16)] = plsc.load_gather(
                            row, [hi]
                        )

                    pltpu.async_copy(
                        outb[b],
                        out_hbm.at[ch, pl.ds(ck * _CHUNK, _CHUNK)],
                        semo[b],
                    )
                return carry2

            lax.fori_loop(0, _KGRP, chunk_grp, 0)

            @pl.when(g < _PAIRS - 1)
            def _(p=p, ch=ch, row=row):
                for c in _row_copies(img_hbm, ch + 2, row, *semg[p]):
                    c.start()

        return carry

    lax.fori_loop(0, _PAIRS, pair_body, 0)

    last_ch = base_ch + _CPW - 1
    for b in range(_NOB):
        pltpu.make_async_copy(
            outb[b],
            out_hbm.at[last_ch, pl.ds((_NCHK - _NOB + b) * _CHUNK, _CHUNK)],
            semo[b],
        ).wait()


@jax.jit
def _shuffle(flat_img, idxp):
    mesh = plsc.VectorSubcoreMesh(core_axis_name="c", subcore_axis_name="s")
    fn = functools.partial(
        pl.kernel,
        mesh=mesh,
        compiler_params=pltpu.CompilerParams(needs_layout_passes=False),
        out_type=jax.ShapeDtypeStruct((C, HW), jnp.float32),
        scratch_types=[
            pltpu.VMEM((HW // 2,), jnp.int32),
            pltpu.VMEM((HW,), jnp.float32),
            pltpu.VMEM((HW,), jnp.float32),
            pltpu.VMEM((_CHUNK,), jnp.float32),
            pltpu.VMEM((_CHUNK,), jnp.float32),
            pltpu.VMEM((_CHUNK,), jnp.float32),
            pltpu.VMEM((_CHUNK,), jnp.float32),
            pltpu.SemaphoreType.DMA,
            pltpu.SemaphoreType.DMA,
            pltpu.SemaphoreType.DMA,
            pltpu.SemaphoreType.DMA,
            pltpu.SemaphoreType.DMA,
            pltpu.SemaphoreType.DMA,
            pltpu.SemaphoreType.DMA,
            pltpu.SemaphoreType.DMA,
        ],
    )(_shuffle_body)
    return fn(flat_img, idxp)


def kernel(img, indices):
    Cc, Hh, Ww = img.shape
    flat = img.reshape(Cc, Hh * Ww)
    idx32 = indices.astype(jnp.int32)
    r = idx32.reshape(HW // 32, 2, 16)
    idxp = (r[:, 0, :] | (r[:, 1, :] << 16)).reshape(HW // 2)
    out = _shuffle(flat, idxp)
    return out.reshape(Cc, Hh, Ww)


# speedup vs baseline: 1.0047x; 1.0047x over previous
import functools

import jax
import jax.numpy as jnp
from jax import lax
from jax.experimental import pallas as pl
from jax.experimental.pallas import tpu as pltpu
from jax.experimental.pallas import tpu_sc as plsc

C, H, W = 768, 224, 224
HW = H * W

_NC = 2
_NS = 16
_NW = _NC * _NS
_CPW = C // _NW
_PAIRS = _CPW // 2

_CHUNK = 1792
_NCHK = HW // _CHUNK
_KPAIRS = _NCHK // 2
_BLKS = _CHUNK // 32


def _shuffle_body(img_hbm, idxp_hbm, out_hbm, idx_v, row0, row1, ob0, ob1,
                  sg0, sg1, so0, so1):
    rows = (row0, row1)
    outb = (ob0, ob1)
    semg = (sg0, sg1)
    semo = (so0, so1)
    wid = lax.axis_index("s") * _NC + lax.axis_index("c")
    base_ch = wid * _CPW

    pltpu.sync_copy(idxp_hbm, idx_v)

    pltpu.async_copy(img_hbm.at[base_ch], row0, sg0)
    pltpu.async_copy(img_hbm.at[base_ch + 1], row1, sg1)

    def pair_body(g, carry):
        for p in (0, 1):
            ch = base_ch + 2 * g + p
            row = rows[p]
            pltpu.make_async_copy(img_hbm.at[ch], row, semg[p]).wait()

            def chunk_pair(k, carry2, p=p, ch=ch, row=row, g=g):
                for b in (0, 1):
                    ck = 2 * k + b

                    def do_wait(b=b, ch=ch, ck=ck):
                        pltpu.make_async_copy(
                            outb[b],
                            out_hbm.at[ch, pl.ds(ck * _CHUNK, _CHUNK)],
                            semo[b],
                        ).wait()

                    if p == 0:
                        pl.when(jnp.logical_or(g > 0, k > 0))(do_wait)
                    else:
                        do_wait()

                    @plsc.parallel_loop(0, _BLKS, unroll=8)
                    def _(t, b=b, ck=ck, row=row):
                        jbase = ck * (_CHUNK // 2) + 16 * t
                        v = idx_v[pl.ds(jbase, 16)]
                        lo = v & 0xFFFF
                        hi = (v >> 16) & 0xFFFF
                        outb[b][pl.ds(32 * t, 16)] = plsc.load_gather(
                            row, [lo]
                        )
                        outb[b][pl.ds(32 * t + 16, 16)] = plsc.load_gather(
                            row, [hi]
                        )

                    pltpu.async_copy(
                        outb[b],
                        out_hbm.at[ch, pl.ds(ck * _CHUNK, _CHUNK)],
                        semo[b],
                    )
                return carry2

            lax.fori_loop(0, _KPAIRS, chunk_pair, 0)

            @pl.when(g < _PAIRS - 1)
            def _(p=p, ch=ch, row=row):
                pltpu.async_copy(img_hbm.at[ch + 2], row, semg[p])

        return carry

    lax.fori_loop(0, _PAIRS, pair_body, 0)

    last_ch = base_ch + _CPW - 1
    for b in (0, 1):
        pltpu.make_async_copy(
            outb[b],
            out_hbm.at[last_ch, pl.ds((_NCHK - 2 + b) * _CHUNK, _CHUNK)],
            semo[b],
        ).wait()


@jax.jit
def _shuffle(flat_img, idxp):
    mesh = plsc.VectorSubcoreMesh(core_axis_name="c", subcore_axis_name="s")
    fn = functools.partial(
        pl.kernel,
        mesh=mesh,
        compiler_params=pltpu.CompilerParams(needs_layout_passes=False),
        out_type=jax.ShapeDtypeStruct((C, HW), jnp.float32),
        scratch_types=[
            pltpu.VMEM((HW // 2,), jnp.int32),
            pltpu.VMEM((HW,), jnp.float32),
            pltpu.VMEM((HW,), jnp.float32),
            pltpu.VMEM((_CHUNK,), jnp.float32),
            pltpu.VMEM((_CHUNK,), jnp.float32),
            pltpu.SemaphoreType.DMA,
            pltpu.SemaphoreType.DMA,
            pltpu.SemaphoreType.DMA,
            pltpu.SemaphoreType.DMA,
        ],
    )(_shuffle_body)
    return fn(flat_img, idxp)


def kernel(img, indices):
    Cc, Hh, Ww = img.shape
    flat = img.reshape(Cc, Hh * Ww)
    idx32 = indices.astype(jnp.int32)
    r = idx32.reshape(HW // 32, 2, 16)
    idxp = (r[:, 0, :] | (r[:, 1, :] << 16)).reshape(HW // 2)
    out = _shuffle(flat, idxp)
    return out.reshape(Cc, Hh, Ww)
